# baseline (speedup 1.0000x reference)
import functools
import math

import jax
import jax.numpy as jnp
from jax import lax
from jax.experimental import pallas as pl
from jax.experimental.pallas import tpu as pltpu
from jax.experimental.pallas import tpu_sc as plsc

N_NODE = 10000
HID = 256
H = 8
DH = 32
NC, NS = 2, 16
NW = NC * NS
INV_SQRT_DH = 1.0 / math.sqrt(DH)
BE = 2000
BM = 1000
SCB = 128
NPAD = 10240
ZR = 128


def _mesh():
    return plsc.VectorSubcoreMesh(core_axis_name="c", subcore_axis_name="s",
                                  num_cores=NC, num_subcores=NS)


def _mm_body(x_ref, w_ref, b_ref, o_ref, *, act, tab3d):
    y = jnp.dot(x_ref[...], w_ref[...], preferred_element_type=jnp.float32)
    y = y + b_ref[...]
    if act == "relu":
        y = jnp.maximum(y, 0.0)
    if tab3d:
        parts = []
        for b in range(y.shape[1] // HID):
            sl = y[:, b * HID:(b + 1) * HID]
            lo = lax.bitcast_convert_type(
                sl[:, :HID // 2].astype(jnp.bfloat16).astype(jnp.float32),
                jnp.int32)
            hi = lax.bitcast_convert_type(
                sl[:, HID // 2:].astype(jnp.bfloat16).astype(jnp.float32),
                jnp.int32)
            parts.append(lax.shift_right_logical(lo, 16) | hi)
        o_ref[...] = jnp.concatenate(parts, axis=1)
    else:
        o_ref[...] = y


@functools.cache
def _mm_call(M, K, Nout, act, tab3d=False):
    if tab3d:
        out_spec = pl.BlockSpec((BM, Nout // 2), lambda i: (i, 0))
        out_shape = jax.ShapeDtypeStruct((M, Nout // 2), jnp.int32)
    else:
        out_spec = pl.BlockSpec((BM, Nout), lambda i: (i, 0))
        out_shape = jax.ShapeDtypeStruct((M, Nout), jnp.float32)
    return pl.pallas_call(
        functools.partial(_mm_body, act=act, tab3d=tab3d),
        grid=(M // BM,),
        in_specs=[pl.BlockSpec((BM, K), lambda i: (i, 0)),
                  pl.BlockSpec((K, Nout), lambda i: (0, 0)),
                  pl.BlockSpec((1, Nout), lambda i: (0, 0))],
        out_specs=out_spec,
        out_shape=out_shape,
    )


def _mm(x, wb, act=None, tab3d=False):
    w, b = wb
    return _mm_call(x.shape[0], x.shape[1], w.shape[1], act, tab3d)(
        x, w, b.reshape(1, -1))


def _unpack_lo(x):
    return lax.bitcast_convert_type(lax.shift_left(x, 16), jnp.float32)


def _unpack_hi(x):
    return lax.bitcast_convert_type(x & jnp.int32(-65536), jnp.float32)


def _alpha_body(q_ref, k_ref, p_ref, a_ref, m_ref):
    q, kk = q_ref[...], k_ref[...]
    pl_ = _unpack_lo(q) * _unpack_lo(kk)
    ph_ = _unpack_hi(q) * _unpack_hi(kk)
    al = jnp.concatenate(
        [pl_.reshape(BE, H // 2, DH).sum(axis=-1),
         ph_.reshape(BE, H // 2, DH).sum(axis=-1)], axis=1)
    al = al * (p_ref[...] * INV_SQRT_DH)
    a_ref[...] = al
    m_ref[...] = jnp.max(al, axis=0, keepdims=True)[None]


@functools.cache
def _alpha_call(E):
    nb = E // BE
    return pl.pallas_call(
        _alpha_body,
        grid=(nb,),
        in_specs=[pl.BlockSpec((BE, HID // 2), lambda i: (i, 0)),
                  pl.BlockSpec((BE, HID // 2), lambda i: (i, 0)),
                  pl.BlockSpec((1, H), lambda i: (0, 0))],
        out_specs=[pl.BlockSpec((BE, H), lambda i: (i, 0)),
                   pl.BlockSpec((1, 1, H), lambda i: (i, 0, 0))],
        out_shape=[jax.ShapeDtypeStruct((E, H), jnp.float32),
                   jax.ShapeDtypeStruct((nb, 1, H), jnp.float32)],
    )


def _msg_body(v_ref, a_ref, pm_ref, d_ref, o_ref, e_ref):
    gm = jnp.max(pm_ref[...], axis=0)
    ex = jnp.exp(a_ref[...] - gm)
    hpc = H // NC
    for c in range(NC):
        vc = (_unpack_lo(v_ref[...]) if c == 0 else _unpack_hi(v_ref[...]))
        ah = ex[:, c * hpc:(c + 1) * hpc]
        o_ref[c] = (vc.reshape(BE, hpc, DH) * ah[:, :, None]).reshape(
            BE, HID // NC)
    sub = (d_ref[...][:, 0] & 7)[:, None]
    lane = lax.broadcasted_iota(jnp.int32, (BE, HID // NC), 1)
    exrep = jnp.tile(jnp.concatenate(
        [ex, jnp.zeros((BE, H), jnp.float32)], axis=1), (1, H))
    e_ref[...] = jnp.where(lane // 16 == sub, exrep, 0.0)


@functools.cache
def _msg_call(E):
    nb = E // BE
    return pl.pallas_call(
        _msg_body,
        grid=(nb,),
        in_specs=[pl.BlockSpec((BE, HID // 2), lambda i: (i, 1)),
                  pl.BlockSpec((BE, H), lambda i: (i, 0)),
                  pl.BlockSpec((nb, 1, H), lambda i: (0, 0, 0)),
                  pl.BlockSpec((BE, 1), lambda i: (i, 0))],
        out_specs=[pl.BlockSpec((NC, BE, HID // NC), lambda i: (0, i, 0)),
                   pl.BlockSpec((BE, HID // NC), lambda i: (i, 0))],
        out_shape=[jax.ShapeDtypeStruct((NC, E, HID // NC), jnp.float32),
                   jax.ShapeDtypeStruct((E, HID // NC), jnp.float32)],
    )


def _epi_body(m_ref, dn_ref, x_ref, w_ref, b_ref, s_ref, o_ref):
    den = dn_ref[0, :, :H] + dn_ref[1, :, :H]
    m = m_ref[...].reshape(BM, H, DH) / (den[:, :, None] + 1e-16)
    o = jax.nn.gelu(m.reshape(BM, HID))
    o = jnp.dot(o, w_ref[...], preferred_element_type=jnp.float32) + b_ref[...]
    sk = jax.nn.sigmoid(s_ref[...])
    o_ref[...] = sk * o + (1.0 - sk) * x_ref[...]


@functools.cache
def _epi_call(M):
    return pl.pallas_call(
        _epi_body,
        grid=(M // BM,),
        in_specs=[pl.BlockSpec((BM, HID), lambda i: (i, 0)),
                  pl.BlockSpec((2, BM, 2 * H), lambda i: (0, i, 0)),
                  pl.BlockSpec((BM, HID), lambda i: (i, 0)),
                  pl.BlockSpec((HID, HID), lambda i: (0, 0)),
                  pl.BlockSpec((1, HID), lambda i: (0, 0)),
                  pl.BlockSpec((1, 1), lambda i: (0, 0))],
        out_specs=pl.BlockSpec((BM, HID), lambda i: (i, 0)),
        out_shape=jax.ShapeDtypeStruct((M, HID), jnp.float32),
    )


def _epi(msg_den, x, awb, skip):
    msg, den = msg_den
    w, b = awb
    return _epi_call(x.shape[0])(msg, den, x, w, b.reshape(1, -1),
                                 skip.reshape(1, 1))


def _final_body(x_ref, w_ref, b_ref, o_ref):
    x = x_ref[...]
    y = jnp.dot(x, w_ref[...], preferred_element_type=jnp.float32) + b_ref[...]
    o_ref[...] = jnp.concatenate(
        [x[:, 1:2], y[:, 0:1], jnp.floor(jnp.abs(y[:, 1:2]))], axis=1)


@functools.cache
def _final_call(M):
    return pl.pallas_call(
        _final_body,
        grid=(M // BM,),
        in_specs=[pl.BlockSpec((BM, HID), lambda i: (i, 0)),
                  pl.BlockSpec((HID, 2), lambda i: (0, 0)),
                  pl.BlockSpec((1, 2), lambda i: (0, 0))],
        out_specs=pl.BlockSpec((BM, 3), lambda i: (i, 0)),
        out_shape=jax.ShapeDtypeStruct((M, 3), jnp.float32),
    )


def _gather_pipe(tab_hbm, out_hbm, idx_v, r0, r1, rt, g0, g1, w0, w1,
                 base_w, iters, tail):
    def g_desc(i, buf, sem):
        return pltpu.make_async_copy(
            tab_hbm.at[idx_v.at[pl.ds(i * SCB, SCB)]], buf, sem)

    def w_desc(i, buf, sem):
        return pltpu.make_async_copy(
            buf, out_hbm.at[pl.ds(base_w + i * SCB, SCB)], sem)

    pairs = iters // 2
    g_desc(0, r0, g0).start()

    def body(j, carry):
        i0 = 2 * j
        g_desc(i0, r0, g0).wait()
        w_desc(i0, r0, w0).start()

        @pl.when(j > 0)
        def _():
            w_desc(i0 - 1, r1, w1).wait()

        g_desc(i0 + 1, r1, g1).start()
        g_desc(i0 + 1, r1, g1).wait()
        w_desc(i0 + 1, r1, w1).start()
        w_desc(i0, r0, w0).wait()

        @pl.when(i0 + 2 < iters)
        def _():
            g_desc(i0 + 2, r0, g0).start()

        return carry

    lax.fori_loop(0, pairs, body, 0)
    if pairs > 0:
        w_desc(2 * pairs - 1, r1, w1).wait()
    if iters % 2:
        i = iters - 1
        g_desc(i, r0, g0).wait()
        w_desc(i, r0, w0).start()
        w_desc(i, r0, w0).wait()
    if tail:
        base = base_w + iters * SCB
        td = pltpu.make_async_copy(
            tab_hbm.at[idx_v.at[pl.ds(iters * SCB, tail)]], rt, g1)
        td.start()
        td.wait()
        pltpu.sync_copy(rt, out_hbm.at[pl.ds(base, tail)])


@functools.cache
def _gather2_call(E):
    per_w = E // NW
    iters = per_w // SCB
    tail = per_w % SCB
    DQ, DKV = HID // 2, HID

    @functools.partial(
        pl.kernel, mesh=_mesh(),
        out_type=[jax.ShapeDtypeStruct((E, DQ), jnp.int32),
                  jax.ShapeDtypeStruct((E, DKV), jnp.int32)],
        scratch_types=[pltpu.VMEM((per_w,), jnp.int32),
                       pltpu.VMEM((SCB, DQ), jnp.int32),
                       pltpu.VMEM((SCB, DQ), jnp.int32),
                       pltpu.VMEM((max(tail, 8), DQ), jnp.int32),
                       pltpu.VMEM((SCB, DKV), jnp.int32),
                       pltpu.VMEM((SCB, DKV), jnp.int32),
                       pltpu.VMEM((max(tail, 8), DKV), jnp.int32),
                       pltpu.SemaphoreType.DMA,
                       pltpu.SemaphoreType.DMA,
                       pltpu.SemaphoreType.DMA,
                       pltpu.SemaphoreType.DMA],
    )
    def k(qtab_hbm, kvtab_hbm, didx_hbm, sidx_hbm, qout_hbm, kvout_hbm,
          idx_v, q0, q1, qt, v0, v1, vt, g0, g1, w0, w1):
        wid = lax.axis_index("s") * NC + lax.axis_index("c")
        base_w = wid * per_w
        pltpu.sync_copy(didx_hbm.at[pl.ds(base_w, per_w)], idx_v)
        _gather_pipe(qtab_hbm, qout_hbm, idx_v, q0, q1, qt, g0, g1, w0, w1,
                     base_w, iters, tail)
        pltpu.sync_copy(sidx_hbm.at[pl.ds(base_w, per_w)], idx_v)
        _gather_pipe(kvtab_hbm, kvout_hbm, idx_v, v0, v1, vt, g0, g1, w0, w1,
                     base_w, iters, tail)

    return k


def _scatter_pipe(src_at, idx_hbm, accum, i0v, i1v, r0, r1, idx_t, rows_t,
                  sems, tile_base, iters, tail):
    is0, is1, ms0, ms1, ss0, ss1 = sems

    def i_desc(i, buf, sem):
        return pltpu.make_async_copy(
            idx_hbm.at[pl.ds(tile_base + i * SCB, SCB)], buf, sem)

    def m_desc(i, buf, sem):
        return pltpu.make_async_copy(src_at(tile_base + i * SCB, SCB), buf,
                                     sem)

    class _Scat:
        def __init__(self, rbuf, ibuf, sem):
            self.rbuf, self.ibuf, self.sem = rbuf, ibuf, sem

        def start(self):
            pltpu.async_copy(self.rbuf, accum.at[self.ibuf], self.sem,
                             add=True)

        def wait(self):
            pltpu.make_async_copy(self.rbuf, accum.at[self.ibuf],
                                  self.sem).wait()

    def s_desc(rbuf, ibuf, sem):
        return _Scat(rbuf, ibuf, sem)

    def load(i, ibuf, rbuf, isem, msem):
        i_desc(i, ibuf, isem).start()
        m_desc(i, rbuf, msem).start()

    load(0, i0v, r0, is0, ms0)

    def body(j, carry):
        i0 = 2 * j
        i_desc(i0, i0v, is0).wait()
        m_desc(i0, r0, ms0).wait()
        s_desc(r0, i0v, ss0).start()

        @pl.when(j > 0)
        def _():
            s_desc(r1, i1v, ss1).wait()

        load(i0 + 1, i1v, r1, is1, ms1)
        i_desc(i0 + 1, i1v, is1).wait()
        m_desc(i0 + 1, r1, ms1).wait()
        s_desc(r1, i1v, ss1).start()
        s_desc(r0, i0v, ss0).wait()

        @pl.when(i0 + 2 < iters)
        def _():
            load(i0 + 2, i0v, r0, is0, ms0)

        return carry

    lax.fori_loop(0, iters // 2, body, 0)
    if iters // 2 > 0:
        s_desc(r1, i1v, ss1).wait()
    if iters % 2:
        i = iters - 1
        i_desc(i, i0v, is0).wait()
        m_desc(i, r0, ms0).wait()
        s_desc(r0, i0v, ss0).start()
        s_desc(r0, i0v, ss0).wait()
    if tail:
        base = tile_base + iters * SCB
        pltpu.sync_copy(idx_hbm.at[pl.ds(base, tail)], idx_t)
        pltpu.sync_copy(src_at(base, tail), rows_t)
        pltpu.sync_copy(rows_t, accum.at[idx_t], add=True)


@functools.cache
def _scatter2_call(E):
    D = HID // NC
    per_tile = E // NS
    iters_m, tail_m = per_tile // SCB, per_tile % SCB
    per_w = E // NW
    iters_d, tail_d = per_w // SCB, per_w % SCB
    rows_pt = NPAD // NS
    dpad = NPAD // 8
    drows_pt = dpad // NS

    @functools.partial(
        pl.kernel, mesh=_mesh(),
        out_type=[jax.ShapeDtypeStruct((NC, NPAD, D), jnp.float32),
                  jax.ShapeDtypeStruct((NC, dpad, D), jnp.float32)],
        scratch_types=[pltpu.VMEM((SCB,), jnp.int32),
                       pltpu.VMEM((SCB,), jnp.int32),
                       pltpu.VMEM((SCB, D), jnp.float32),
                       pltpu.VMEM((SCB, D), jnp.float32),
                       pltpu.VMEM((max(tail_m, 8),), jnp.int32),
                       pltpu.VMEM((max(tail_m, 8), D), jnp.float32),
                       pltpu.VMEM((max(tail_d, 8),), jnp.int32),
                       pltpu.VMEM((max(tail_d, 8), D), jnp.float32),
                       pltpu.VMEM_SHARED((NPAD, D), jnp.float32),
                       pltpu.VMEM_SHARED((dpad, D), jnp.float32),
                       pltpu.SemaphoreType.DMA,
                       pltpu.SemaphoreType.DMA,
                       pltpu.SemaphoreType.DMA,
                       pltpu.SemaphoreType.DMA,
                       pltpu.SemaphoreType.DMA,
                       pltpu.SemaphoreType.DMA],
    )
    def k(msg_hbm, exp_hbm, didx_hbm, didx8_hbm, zero_hbm, out_hbm, den_hbm,
          i0v, i1v, r0, r1, idx_tm, rows_tm, idx_td, rows_td, accum, daccum,
          is0, is1, ms0, ms1, ss0, ss1):
        c = lax.axis_index("c")
        s = lax.axis_index("s")
        sems = (is0, is1, ms0, ms1, ss0, ss1)
        pltpu.sync_copy(zero_hbm, r0.at[pl.ds(0, ZR)])

        def zb(z, carry):
            pltpu.sync_copy(r0.at[pl.ds(0, ZR)],
                            accum.at[pl.ds(s * rows_pt + z * ZR, ZR)])
            return carry

        lax.fori_loop(0, rows_pt // ZR, zb, 0)
        pltpu.sync_copy(r0.at[pl.ds(0, drows_pt)],
                        daccum.at[pl.ds(s * drows_pt, drows_pt)])
        plsc.subcore_barrier()

        _scatter_pipe(lambda b, n: msg_hbm.at[c, pl.ds(b, n)], didx_hbm,
                      accum, i0v, i1v, r0, r1, idx_tm, rows_tm, sems,
                      s * per_tile, iters_m, tail_m)
        _scatter_pipe(lambda b, n: exp_hbm.at[pl.ds(b, n)], didx8_hbm,
                      daccum, i0v, i1v, r0, r1, idx_td, rows_td, sems,
                      (c * NS + s) * per_w, iters_d, tail_d)

        plsc.subcore_barrier()
        pltpu.sync_copy(accum.at[pl.ds(s * rows_pt, rows_pt)],
                        out_hbm.at[c, pl.ds(s * rows_pt, rows_pt)])
        pltpu.sync_copy(daccum.at[pl.ds(s * drows_pt, drows_pt)],
                        den_hbm.at[c, pl.ds(s * drows_pt, drows_pt)])

    return k


def _fold_kv(wb, rel):
    w, b = wb
    wf = jnp.einsum('fhd,hde->fhe', w.reshape(HID, H, DH), rel).reshape(HID, HID)
    bf = jnp.einsum('hd,hde->he', b.reshape(H, DH), rel).reshape(HID)
    return wf, bf


def _kv_wb(lp, t, ek):
    wk, bk = _fold_kv(lp['k'][t], lp['rel'][ek]['a_rel'])
    wv, bv = _fold_kv(lp['v'][t], lp['rel'][ek]['m_rel'])
    return (jnp.concatenate([wk, wv], axis=1),
            jnp.concatenate([bk, bv]))


def _edge_pass(q_tab, kv_tab, p_rel, sidx, didx, didx8, z128):
    E = sidx.shape[0]
    qg, kvg = _gather2_call(E)(q_tab, kv_tab, didx, sidx)
    alpha, pmax = _alpha_call(E)(qg, kvg, p_rel.reshape(1, H))
    msg2, ex128 = _msg_call(E)(kvg, alpha, pmax, didx.reshape(E, 1))
    out2, den2 = _scatter2_call(E)(msg2, ex128, didx, didx8, z128)
    den = den2.reshape(NC, NPAD, 2 * H)
    return (jnp.concatenate([out2[0, :N_NODE], out2[1, :N_NODE]], axis=1),
            den[:, :N_NODE])


def kernel(x_ue, x_ap, params, edge_index_ue_to_ap, edge_index_ap_to_ue):
    p = params
    s_ua = edge_index_ue_to_ap[0].astype(jnp.int32)
    d_ua = edge_index_ue_to_ap[1].astype(jnp.int32)
    s_au = edge_index_ap_to_ue[0].astype(jnp.int32)
    d_au = edge_index_ap_to_ue[1].astype(jnp.int32)
    d_ua8 = d_ua // 8
    d_au8 = d_au // 8
    z128 = jnp.zeros((ZR, HID // NC), jnp.float32)

    x = {'ue': _mm(x_ue, p['in']['ue'], act="relu"),
         'ap': _mm(x_ap, p['in']['ap'], act="relu")}

    lp = p['layers'][0]
    q_ue = _mm(x['ue'], lp['q']['ue'], tab3d=True)
    q_ap = _mm(x['ap'], lp['q']['ap'], tab3d=True)
    kv_ue = _mm(x['ue'], _kv_wb(lp, 'ue', 'ue__ap'), tab3d=True)
    kv_ap = _mm(x['ap'], _kv_wb(lp, 'ap', 'ap__ue'), tab3d=True)
    msg_ap = _edge_pass(q_ap, kv_ue, lp['rel']['ue__ap']['p_rel'],
                        s_ua, d_ua, d_ua8, z128)
    msg_ue = _edge_pass(q_ue, kv_ap, lp['rel']['ap__ue']['p_rel'],
                        s_au, d_au, d_au8, z128)
    x = {'ue': _epi(msg_ue, x['ue'], lp['a']['ue'], lp['skip']['ue']),
         'ap': _epi(msg_ap, x['ap'], lp['a']['ap'], lp['skip']['ap'])}

    lp = p['layers'][1]
    q_ue2 = _mm(x['ue'], lp['q']['ue'], tab3d=True)
    kv_ap2 = _mm(x['ap'], _kv_wb(lp, 'ap', 'ap__ue'), tab3d=True)
    msg_ue2 = _edge_pass(q_ue2, kv_ap2, lp['rel']['ap__ue']['p_rel'],
                         s_au, d_au, d_au8, z128)
    x_ue_out = _epi(msg_ue2, x['ue'], lp['a']['ue'], lp['skip']['ue'])

    wf = jnp.stack([p['lin'][0][:, 1], p['lin1'][0][:, 1]], axis=1)
    bf = jnp.stack([p['lin'][1][1], p['lin1'][1][1]]).reshape(1, 2)
    return _final_call(N_NODE)(x_ue_out, wf, bf)


# speedup vs baseline: 1.1609x; 1.1609x over previous
import functools
import math

import jax
import jax.numpy as jnp
from jax import lax
from jax.experimental import pallas as pl
from jax.experimental.pallas import tpu as pltpu
from jax.experimental.pallas import tpu_sc as plsc

N_NODE = 10000
HID = 256
H = 8
DH = 32
NC, NS = 2, 16
NW = NC * NS
INV_SQRT_DH = 1.0 / math.sqrt(DH)
BE = 2000
BM = 1000
SCB = 128
NPAD = 10240
ZR = 128


def _mesh():
    return plsc.VectorSubcoreMesh(core_axis_name="c", subcore_axis_name="s",
                                  num_cores=NC, num_subcores=NS)


def _mm_body(x_ref, w_ref, b_ref, o_ref, *, act, tab3d):
    y = jnp.dot(x_ref[...], w_ref[...], preferred_element_type=jnp.float32)
    y = y + b_ref[...]
    if act == "relu":
        y = jnp.maximum(y, 0.0)
    if tab3d:
        parts = []
        for b in range(y.shape[1] // HID):
            sl = y[:, b * HID:(b + 1) * HID]
            lo = lax.bitcast_convert_type(
                sl[:, :HID // 2].astype(jnp.bfloat16).astype(jnp.float32),
                jnp.int32)
            hi = lax.bitcast_convert_type(
                sl[:, HID // 2:].astype(jnp.bfloat16).astype(jnp.float32),
                jnp.int32)
            parts.append(lax.shift_right_logical(lo, 16) | hi)
        o_ref[...] = jnp.concatenate(parts, axis=1)
    else:
        o_ref[...] = y


@functools.cache
def _mm_call(M, K, Nout, act, tab3d=False):
    if tab3d:
        out_spec = pl.BlockSpec((BM, Nout // 2), lambda i: (i, 0))
        out_shape = jax.ShapeDtypeStruct((M, Nout // 2), jnp.int32)
    else:
        out_spec = pl.BlockSpec((BM, Nout), lambda i: (i, 0))
        out_shape = jax.ShapeDtypeStruct((M, Nout), jnp.float32)
    return pl.pallas_call(
        functools.partial(_mm_body, act=act, tab3d=tab3d),
        grid=(M // BM,),
        in_specs=[pl.BlockSpec((BM, K), lambda i: (i, 0)),
                  pl.BlockSpec((K, Nout), lambda i: (0, 0)),
                  pl.BlockSpec((1, Nout), lambda i: (0, 0))],
        out_specs=out_spec,
        out_shape=out_shape,
    )


def _mm(x, wb, act=None, tab3d=False):
    w, b = wb
    return _mm_call(x.shape[0], x.shape[1], w.shape[1], act, tab3d)(
        x, w, b.reshape(1, -1))


def _unpack_lo(x):
    return lax.bitcast_convert_type(lax.shift_left(x, 16), jnp.float32)


def _unpack_hi(x):
    return lax.bitcast_convert_type(x & jnp.int32(-65536), jnp.float32)


def _alpha_body(q_ref, k_ref, p_ref, a_ref, m_ref):
    q, kk = q_ref[...], k_ref[...]
    pl_ = _unpack_lo(q) * _unpack_lo(kk)
    ph_ = _unpack_hi(q) * _unpack_hi(kk)
    al = jnp.concatenate(
        [pl_.reshape(BE, H // 2, DH).sum(axis=-1),
         ph_.reshape(BE, H // 2, DH).sum(axis=-1)], axis=1)
    al = al * (p_ref[...] * INV_SQRT_DH)
    a_ref[...] = al
    m_ref[...] = jnp.max(al, axis=0, keepdims=True)[None]


@functools.cache
def _alpha_call(E):
    nb = E // BE
    return pl.pallas_call(
        _alpha_body,
        grid=(nb,),
        in_specs=[pl.BlockSpec((BE, HID // 2), lambda i: (i, 0)),
                  pl.BlockSpec((BE, HID // 2), lambda i: (i, 0)),
                  pl.BlockSpec((1, H), lambda i: (0, 0))],
        out_specs=[pl.BlockSpec((BE, H), lambda i: (i, 0)),
                   pl.BlockSpec((1, 1, H), lambda i: (i, 0, 0))],
        out_shape=[jax.ShapeDtypeStruct((E, H), jnp.float32),
                   jax.ShapeDtypeStruct((nb, 1, H), jnp.float32)],
    )


def _msg_body(v_ref, a_ref, pm_ref, o_ref, e_ref):
    gm = jnp.max(pm_ref[...], axis=0)
    ex = jnp.exp(a_ref[...] - gm)
    hpc = H // NC
    for c in range(NC):
        vc = (_unpack_lo(v_ref[...]) if c == 0 else _unpack_hi(v_ref[...]))
        ah = ex[:, c * hpc:(c + 1) * hpc]
        o_ref[c] = (vc.reshape(BE, hpc, DH) * ah[:, :, None]).reshape(
            BE, HID // NC)
    e_ref[...] = jnp.concatenate(
        [ex, jnp.zeros((BE, HID // NC - H), jnp.float32)], axis=1)


@functools.cache
def _msg_call(E):
    nb = E // BE
    return pl.pallas_call(
        _msg_body,
        grid=(nb,),
        in_specs=[pl.BlockSpec((BE, HID // 2), lambda i: (i, 1)),
                  pl.BlockSpec((BE, H), lambda i: (i, 0)),
                  pl.BlockSpec((nb, 1, H), lambda i: (0, 0, 0))],
        out_specs=[pl.BlockSpec((NC, BE, HID // NC), lambda i: (0, i, 0)),
                   pl.BlockSpec((BE, HID // NC), lambda i: (i, 0))],
        out_shape=[jax.ShapeDtypeStruct((NC, E, HID // NC), jnp.float32),
                   jax.ShapeDtypeStruct((E, HID // NC), jnp.float32)],
    )


def _epi_body(m_ref, dn_ref, x_ref, w_ref, b_ref, s_ref, o_ref):
    den = jnp.maximum(dn_ref[0, :, :H] + dn_ref[1, :, :H], 1e-30)
    m = m_ref[...].reshape(BM, H, DH) / den[:, :, None]
    o = jax.nn.gelu(m.reshape(BM, HID))
    o = jnp.dot(o, w_ref[...], preferred_element_type=jnp.float32) + b_ref[...]
    sk = jax.nn.sigmoid(s_ref[...])
    o_ref[...] = sk * o + (1.0 - sk) * x_ref[...]


@functools.cache
def _epi_call(M):
    return pl.pallas_call(
        _epi_body,
        grid=(M // BM,),
        in_specs=[pl.BlockSpec((BM, HID), lambda i: (i, 0)),
                  pl.BlockSpec((2, BM, 2 * H), lambda i: (0, i, 0)),
                  pl.BlockSpec((BM, HID), lambda i: (i, 0)),
                  pl.BlockSpec((HID, HID), lambda i: (0, 0)),
                  pl.BlockSpec((1, HID), lambda i: (0, 0)),
                  pl.BlockSpec((1, 1), lambda i: (0, 0))],
        out_specs=pl.BlockSpec((BM, HID), lambda i: (i, 0)),
        out_shape=jax.ShapeDtypeStruct((M, HID), jnp.float32),
    )


def _epi(msg_den, x, awb, skip):
    msg, den = msg_den
    w, b = awb
    return _epi_call(x.shape[0])(msg, den, x, w, b.reshape(1, -1),
                                 skip.reshape(1, 1))


def _final_body(x_ref, w_ref, b_ref, o_ref):
    x = x_ref[...]
    y = jnp.dot(x, w_ref[...], preferred_element_type=jnp.float32) + b_ref[...]
    o_ref[...] = jnp.concatenate(
        [x[:, 1:2], y[:, 0:1], jnp.floor(jnp.abs(y[:, 1:2]))], axis=1)


@functools.cache
def _final_call(M):
    return pl.pallas_call(
        _final_body,
        grid=(M // BM,),
        in_specs=[pl.BlockSpec((BM, HID), lambda i: (i, 0)),
                  pl.BlockSpec((HID, 2), lambda i: (0, 0)),
                  pl.BlockSpec((1, 2), lambda i: (0, 0))],
        out_specs=pl.BlockSpec((BM, 3), lambda i: (i, 0)),
        out_shape=jax.ShapeDtypeStruct((M, 3), jnp.float32),
    )


def _gather_pipe(tab_hbm, out_hbm, idx_v, r0, r1, rt, g0, g1, w0, w1,
                 base_w, iters, tail):
    def g_desc(i, buf, sem):
        return pltpu.make_async_copy(
            tab_hbm.at[idx_v.at[pl.ds(i * SCB, SCB)]], buf, sem)

    def w_desc(i, buf, sem):
        return pltpu.make_async_copy(
            buf, out_hbm.at[pl.ds(base_w + i * SCB, SCB)], sem)

    pairs = iters // 2
    g_desc(0, r0, g0).start()

    def body(j, carry):
        i0 = 2 * j
        g_desc(i0, r0, g0).wait()
        w_desc(i0, r0, w0).start()

        @pl.when(j > 0)
        def _():
            w_desc(i0 - 1, r1, w1).wait()

        g_desc(i0 + 1, r1, g1).start()
        g_desc(i0 + 1, r1, g1).wait()
        w_desc(i0 + 1, r1, w1).start()
        w_desc(i0, r0, w0).wait()

        @pl.when(i0 + 2 < iters)
        def _():
            g_desc(i0 + 2, r0, g0).start()

        return carry

    lax.fori_loop(0, pairs, body, 0)
    if pairs > 0:
        w_desc(2 * pairs - 1, r1, w1).wait()
    if iters % 2:
        i = iters - 1
        g_desc(i, r0, g0).wait()
        w_desc(i, r0, w0).start()
        w_desc(i, r0, w0).wait()
    if tail:
        base = base_w + iters * SCB
        td = pltpu.make_async_copy(
            tab_hbm.at[idx_v.at[pl.ds(iters * SCB, tail)]], rt, g1)
        td.start()
        td.wait()
        pltpu.sync_copy(rt, out_hbm.at[pl.ds(base, tail)])


@functools.cache
def _gather2_call(E):
    per_w = E // NW
    iters = per_w // SCB
    tail = per_w % SCB
    DQ, DKV = HID // 2, HID

    @functools.partial(
        pl.kernel, mesh=_mesh(),
        out_type=[jax.ShapeDtypeStruct((E, DQ), jnp.int32),
                  jax.ShapeDtypeStruct((E, DKV), jnp.int32)],
        scratch_types=[pltpu.VMEM((per_w,), jnp.int32),
                       pltpu.VMEM((SCB, DQ), jnp.int32),
                       pltpu.VMEM((SCB, DQ), jnp.int32),
                       pltpu.VMEM((max(tail, 8), DQ), jnp.int32),
                       pltpu.VMEM((SCB, DKV), jnp.int32),
                       pltpu.VMEM((SCB, DKV), jnp.int32),
                       pltpu.VMEM((max(tail, 8), DKV), jnp.int32),
                       pltpu.SemaphoreType.DMA,
                       pltpu.SemaphoreType.DMA,
                       pltpu.SemaphoreType.DMA,
                       pltpu.SemaphoreType.DMA],
    )
    def k(qtab_hbm, kvtab_hbm, didx_hbm, sidx_hbm, qout_hbm, kvout_hbm,
          idx_v, q0, q1, qt, v0, v1, vt, g0, g1, w0, w1):
        wid = lax.axis_index("s") * NC + lax.axis_index("c")
        base_w = wid * per_w
        pltpu.sync_copy(didx_hbm.at[pl.ds(base_w, per_w)], idx_v)
        _gather_pipe(qtab_hbm, qout_hbm, idx_v, q0, q1, qt, g0, g1, w0, w1,
                     base_w, iters, tail)
        pltpu.sync_copy(sidx_hbm.at[pl.ds(base_w, per_w)], idx_v)
        _gather_pipe(kvtab_hbm, kvout_hbm, idx_v, v0, v1, vt, g0, g1, w0, w1,
                     base_w, iters, tail)

    return k


@functools.cache
def _scatter_call(E, D, colsplit):
    per_tile = (E // NS) if colsplit else (E // NW)
    iters = per_tile // SCB
    tail = per_tile % SCB
    pairs = iters // 2
    rows_pt = NPAD // NS

    @functools.partial(
        pl.kernel, mesh=_mesh(),
        out_type=jax.ShapeDtypeStruct((NC, NPAD, D), jnp.float32),
        scratch_types=[pltpu.VMEM((SCB,), jnp.int32),
                       pltpu.VMEM((SCB,), jnp.int32),
                       pltpu.VMEM((SCB, D), jnp.float32),
                       pltpu.VMEM((SCB, D), jnp.float32),
                       pltpu.VMEM((max(tail, 8),), jnp.int32),
                       pltpu.VMEM((max(tail, 8), D), jnp.float32),
                       pltpu.VMEM_SHARED((NPAD, D), jnp.float32),
                       pltpu.SemaphoreType.DMA,
                       pltpu.SemaphoreType.DMA,
                       pltpu.SemaphoreType.DMA,
                       pltpu.SemaphoreType.DMA,
                       pltpu.SemaphoreType.DMA,
                       pltpu.SemaphoreType.DMA],
    )
    def k(msg_hbm, didx_hbm, zero_hbm, out_hbm, i0v, i1v, r0, r1, idx_t,
          rows_t, accum, is0, is1, ms0, ms1, ss0, ss1):
        c = lax.axis_index("c")
        s = lax.axis_index("s")
        if colsplit:
            tile_base = s * per_tile
        else:
            tile_base = (c * NS + s) * per_tile
        pltpu.sync_copy(zero_hbm, r0.at[pl.ds(0, ZR)])

        def zb(z, carry):
            pltpu.sync_copy(r0.at[pl.ds(0, ZR)],
                            accum.at[pl.ds(s * rows_pt + z * ZR, ZR)])
            return carry

        lax.fori_loop(0, rows_pt // ZR, zb, 0)
        plsc.subcore_barrier()

        def i_desc(i, buf, sem):
            return pltpu.make_async_copy(
                didx_hbm.at[pl.ds(tile_base + i * SCB, SCB)], buf, sem)

        def m_desc(i, buf, sem):
            if colsplit:
                src = msg_hbm.at[c, pl.ds(tile_base + i * SCB, SCB)]
            else:
                src = msg_hbm.at[pl.ds(tile_base + i * SCB, SCB)]
            return pltpu.make_async_copy(src, buf, sem)

        class _Scat:
            def __init__(self, rbuf, ibuf, sem):
                self.rbuf, self.ibuf, self.sem = rbuf, ibuf, sem

            def start(self):
                pltpu.async_copy(self.rbuf, accum.at[self.ibuf], self.sem,
                                 add=True)

            def wait(self):
                pltpu.make_async_copy(self.rbuf, accum.at[self.ibuf],
                                      self.sem).wait()

        def s_desc(rbuf, ibuf, sem):
            return _Scat(rbuf, ibuf, sem)

        def load(i, ibuf, rbuf, isem, msem):
            i_desc(i, ibuf, isem).start()
            m_desc(i, rbuf, msem).start()

        load(0, i0v, r0, is0, ms0)

        def body(j, carry):
            i0 = 2 * j
            i_desc(i0, i0v, is0).wait()
            m_desc(i0, r0, ms0).wait()
            s_desc(r0, i0v, ss0).start()

            @pl.when(j > 0)
            def _():
                s_desc(r1, i1v, ss1).wait()

            load(i0 + 1, i1v, r1, is1, ms1)
            i_desc(i0 + 1, i1v, is1).wait()
            m_desc(i0 + 1, r1, ms1).wait()
            s_desc(r1, i1v, ss1).start()
            s_desc(r0, i0v, ss0).wait()

            @pl.when(i0 + 2 < iters)
            def _():
                load(i0 + 2, i0v, r0, is0, ms0)

            return carry

        lax.fori_loop(0, pairs, body, 0)
        if pairs > 0:
            s_desc(r1, i1v, ss1).wait()
        if iters % 2:
            i = iters - 1
            i_desc(i, i0v, is0).wait()
            m_desc(i, r0, ms0).wait()
            s_desc(r0, i0v, ss0).start()
            s_desc(r0, i0v, ss0).wait()
        if tail:
            base = tile_base + iters * SCB
            pltpu.sync_copy(didx_hbm.at[pl.ds(base, tail)], idx_t)
            if colsplit:
                pltpu.sync_copy(msg_hbm.at[c, pl.ds(base, tail)], rows_t)
            else:
                pltpu.sync_copy(msg_hbm.at[pl.ds(base, tail)], rows_t)
            pltpu.sync_copy(rows_t, accum.at[idx_t], add=True)
        plsc.subcore_barrier()
        pltpu.sync_copy(accum.at[pl.ds(s * rows_pt, rows_pt)],
                        out_hbm.at[c, pl.ds(s * rows_pt, rows_pt)])

    return k


def _fold_kv(wb, rel):
    w, b = wb
    wf = jnp.einsum('fhd,hde->fhe', w.reshape(HID, H, DH), rel).reshape(HID, HID)
    bf = jnp.einsum('hd,hde->he', b.reshape(H, DH), rel).reshape(HID)
    return wf, bf


def _kv_wb(lp, t, ek):
    wk, bk = _fold_kv(lp['k'][t], lp['rel'][ek]['a_rel'])
    wv, bv = _fold_kv(lp['v'][t], lp['rel'][ek]['m_rel'])
    return (jnp.concatenate([wk, wv], axis=1),
            jnp.concatenate([bk, bv]))


def _edge_pass(q_tab, kv_tab, p_rel, sidx, didx, z128):
    E = sidx.shape[0]
    qg, kvg = _gather2_call(E)(q_tab, kv_tab, didx, sidx)
    alpha, pmax = _alpha_call(E)(qg, kvg, p_rel.reshape(1, H))
    msg2, ex128 = _msg_call(E)(kvg, alpha, pmax)
    den2 = _scatter_call(E, HID // NC, False)(ex128, didx, z128)
    out2 = _scatter_call(E, HID // NC, True)(msg2, didx, z128)
    return (jnp.concatenate([out2[0, :N_NODE], out2[1, :N_NODE]], axis=1),
            den2[:, :N_NODE, :2 * H])


def kernel(x_ue, x_ap, params, edge_index_ue_to_ap, edge_index_ap_to_ue):
    p = params
    s_ua = edge_index_ue_to_ap[0].astype(jnp.int32)
    d_ua = edge_index_ue_to_ap[1].astype(jnp.int32)
    s_au = edge_index_ap_to_ue[0].astype(jnp.int32)
    d_au = edge_index_ap_to_ue[1].astype(jnp.int32)
    z128 = jnp.zeros((ZR, HID // NC), jnp.float32)

    x = {'ue': _mm(x_ue, p['in']['ue'], act="relu"),
         'ap': _mm(x_ap, p['in']['ap'], act="relu")}

    lp = p['layers'][0]
    q_ue = _mm(x['ue'], lp['q']['ue'], tab3d=True)
    q_ap = _mm(x['ap'], lp['q']['ap'], tab3d=True)
    kv_ue = _mm(x['ue'], _kv_wb(lp, 'ue', 'ue__ap'), tab3d=True)
    kv_ap = _mm(x['ap'], _kv_wb(lp, 'ap', 'ap__ue'), tab3d=True)
    msg_ap = _edge_pass(q_ap, kv_ue, lp['rel']['ue__ap']['p_rel'],
                        s_ua, d_ua, z128)
    msg_ue = _edge_pass(q_ue, kv_ap, lp['rel']['ap__ue']['p_rel'],
                        s_au, d_au, z128)
    x = {'ue': _epi(msg_ue, x['ue'], lp['a']['ue'], lp['skip']['ue']),
         'ap': _epi(msg_ap, x['ap'], lp['a']['ap'], lp['skip']['ap'])}

    lp = p['layers'][1]
    q_ue2 = _mm(x['ue'], lp['q']['ue'], tab3d=True)
    kv_ap2 = _mm(x['ap'], _kv_wb(lp, 'ap', 'ap__ue'), tab3d=True)
    msg_ue2 = _edge_pass(q_ue2, kv_ap2, lp['rel']['ap__ue']['p_rel'],
                         s_au, d_au, z128)
    x_ue_out = _epi(msg_ue2, x['ue'], lp['a']['ue'], lp['skip']['ue'])

    wf = jnp.stack([p['lin'][0][:, 1], p['lin1'][0][:, 1]], axis=1)
    bf = jnp.stack([p['lin'][1][1], p['lin1'][1][1]]).reshape(1, 2)
    return _final_call(N_NODE)(x_ue_out, wf, bf)
